# baseline (speedup 1.0000x reference)
import functools

import jax
import jax.numpy as jnp
from jax import lax
from jax.experimental import pallas as pl
from jax.experimental.pallas import tpu as pltpu
from jax.experimental.pallas import tpu_sc as plsc

N = 10000
E = 320000
D = 128
NC = 2
NS = 16
CHUNK = 128
E_PAD = 327680
NCH_T = 80
NROW = E_PAD // CHUNK
N_PAD = 10240
SLICE = N_PAD // NS
ZB = 64
CNTW = 128
G = 8
NCHG = 16

_mesh = plsc.VectorSubcoreMesh(core_axis_name="c", subcore_axis_name="s")


def _fill2d(ref, rows, cols, value):
    val = jnp.full((1, 16), value, jnp.float32)

    @pl.loop(0, rows)
    def _(r):
        @pl.loop(0, cols, step=16)
        def _(j):
            ref[pl.ds(r, 1), pl.ds(j, 16)] = val


def _make_sc_pass():
    out_types = [jax.ShapeDtypeStruct((NC * N_PAD, D), jnp.float32)]
    scratch = [
        pltpu.VMEM_SHARED((N_PAD, D), jnp.float32),
        pltpu.VMEM((NCHG, CHUNK), jnp.int32),
        pltpu.VMEM((NCHG, CHUNK), jnp.int32),
        pltpu.VMEM((CHUNK, D), jnp.float32),
        pltpu.VMEM((CHUNK, D), jnp.float32),
        pltpu.VMEM((ZB, D), jnp.float32),
        pltpu.SemaphoreType.DMA,
        pltpu.SemaphoreType.DMA,
        pltpu.SemaphoreType.DMA,
        pltpu.SemaphoreType.DMA,
    ]

    def body(table_hbm, src_hbm, dst_hbm, acc_out, acc_sh, isrc, idst,
             rows0, rows1, zero_v, gs0, gs1, ss0, ss1):
        c = lax.axis_index("c")
        s = lax.axis_index("s")
        wid = c * NS + s

        _fill2d(zero_v, ZB, D, 0.0)
        tile_base = s * SLICE

        @pl.loop(0, SLICE, step=ZB)
        def _(r):
            pltpu.sync_copy(zero_v, acc_sh.at[pl.ds(tile_base + r, ZB)])

        plsc.subcore_barrier()

        @pl.loop(0, NCH_T // NCHG)
        def _(g):
            row0 = wid * NCH_T + g * NCHG
            pltpu.sync_copy(src_hbm.at[pl.ds(row0, NCHG)], isrc)
            pltpu.sync_copy(dst_hbm.at[pl.ds(row0, NCHG)], idst)
            pltpu.async_copy(table_hbm.at[isrc.at[0]], rows0, gs0)

            @pl.loop(0, NCHG, step=2)
            def _(k):
                @pl.when(k > 0)
                def _():
                    pltpu.make_async_copy(
                        rows1, acc_sh.at[idst.at[k - 1]], ss1).wait()

                @pl.when(k + 1 < NCHG)
                def _():
                    pltpu.async_copy(table_hbm.at[isrc.at[k + 1]], rows1, gs1)

                pltpu.make_async_copy(
                    table_hbm.at[isrc.at[k]], rows0, gs0).wait()
                pltpu.async_copy(rows0, acc_sh.at[idst.at[k]], ss0, add=True)

                pltpu.make_async_copy(
                    rows0, acc_sh.at[idst.at[k]], ss0).wait()

                @pl.when(k + 2 < NCHG)
                def _():
                    pltpu.async_copy(table_hbm.at[isrc.at[k + 2]], rows0, gs0)

                pltpu.make_async_copy(
                    table_hbm.at[isrc.at[k + 1]], rows1, gs1).wait()
                pltpu.async_copy(rows1, acc_sh.at[idst.at[k + 1]], ss1,
                                 add=True)

            pltpu.make_async_copy(
                rows1, acc_sh.at[idst.at[NCHG - 1]], ss1).wait()

        plsc.subcore_barrier()

        pltpu.sync_copy(acc_sh.at[pl.ds(tile_base, SLICE)],
                        acc_out.at[pl.ds(c * N_PAD + tile_base, SLICE)])

    return pl.kernel(body, out_type=out_types, mesh=_mesh,
                     scratch_types=scratch)


def _make_sc_counts():
    out_types = [jax.ShapeDtypeStruct((NC * N_PAD, CNTW), jnp.float32)]
    scratch = [
        pltpu.VMEM_SHARED((N_PAD, CNTW), jnp.float32),
        pltpu.VMEM((G, CHUNK), jnp.int32),
        pltpu.VMEM((CHUNK, CNTW), jnp.float32),
        pltpu.VMEM((ZB, CNTW), jnp.float32),
        pltpu.SemaphoreType.DMA,
    ]

    def body(dst_hbm, cnt_out, cnt_sh, idx_dst, ones_v, zero_v, sem):
        c = lax.axis_index("c")
        s = lax.axis_index("s")
        wid = c * NS + s

        _fill2d(zero_v, ZB, CNTW, 0.0)
        _fill2d(ones_v, CHUNK, CNTW, 1.0)
        tile_base = s * SLICE

        @pl.loop(0, SLICE, step=ZB)
        def _(r):
            pltpu.sync_copy(zero_v, cnt_sh.at[pl.ds(tile_base + r, ZB)])

        plsc.subcore_barrier()

        @pl.loop(0, NCH_T // G)
        def _(g):
            row0 = wid * NCH_T + g * G
            pltpu.sync_copy(dst_hbm.at[pl.ds(row0, G)], idx_dst)

            @pl.loop(0, G)
            def _(i):
                pltpu.sync_copy(ones_v, cnt_sh.at[idx_dst.at[i]], add=True)

        plsc.subcore_barrier()

        pltpu.sync_copy(cnt_sh.at[pl.ds(tile_base, SLICE)],
                        cnt_out.at[pl.ds(c * N_PAD + tile_base, SLICE)])

    return pl.kernel(body, out_type=out_types, mesh=_mesh,
                     scratch_types=scratch)


_sc_agg = _make_sc_pass()
_sc_counts = _make_sc_counts()


_RB = 1000
_GRID = N // _RB


def _dot(a, b):
    return jax.lax.dot_general(a, b, (((1,), (0,)), ((), ())),
                               preferred_element_type=jnp.float32,
                               precision=jax.lax.Precision.HIGHEST)


def _lin_body(x_ref, w_ref, b_ref, o_ref):
    o_ref[...] = _dot(x_ref[...], w_ref[...]) + b_ref[...]


def _tc_linear(x, w, b):
    return pl.pallas_call(
        _lin_body,
        grid=(_GRID,),
        in_specs=[
            pl.BlockSpec((_RB, D), lambda i: (i, 0)),
            pl.BlockSpec((D, D), lambda i: (0, 0)),
            pl.BlockSpec((D,), lambda i: (0,)),
        ],
        out_specs=pl.BlockSpec((_RB, D), lambda i: (i, 0)),
        out_shape=jax.ShapeDtypeStruct((N, D), jnp.float32),
    )(x, w, b)


def _h1_body(a0_ref, a1_ref, c0_ref, c1_ref, wl_ref, r1_ref, h1_ref):
    cnt = c0_ref[...][:, 0:1] + c1_ref[...][:, 0:1]
    mean = (a0_ref[...] + a1_ref[...]) / jnp.maximum(cnt, 1.0)
    h1_ref[...] = jax.nn.relu(_dot(mean, wl_ref[...]) + r1_ref[...])


def _tc_h1(a0, a1, c0, c1, W_l1, r1):
    return pl.pallas_call(
        _h1_body,
        grid=(_GRID,),
        in_specs=[
            pl.BlockSpec((_RB, D), lambda i: (i, 0)),
            pl.BlockSpec((_RB, D), lambda i: (i, 0)),
            pl.BlockSpec((_RB, CNTW), lambda i: (i, 0)),
            pl.BlockSpec((_RB, CNTW), lambda i: (i, 0)),
            pl.BlockSpec((D, D), lambda i: (0, 0)),
            pl.BlockSpec((_RB, D), lambda i: (i, 0)),
        ],
        out_specs=pl.BlockSpec((_RB, D), lambda i: (i, 0)),
        out_shape=jax.ShapeDtypeStruct((N, D), jnp.float32),
    )(a0, a1, c0, c1, W_l1, r1)


def _combine2_body(q0_ref, q1_ref, c0_ref, c1_ref, wl_ref, r2_ref, wp_ref,
                   bp_ref, wd_ref, bd_ref, lo_ref, hi_ref):
    cnt = c0_ref[...][:, 0:1] + c1_ref[...][:, 0:1]
    mean = (q0_ref[...] + q1_ref[...]) / jnp.maximum(cnt, 1.0)
    h2 = jax.nn.relu(_dot(mean, wl_ref[...]) + r2_ref[...])
    preds = _dot(h2, wp_ref[...]) + bp_ref[...]
    diffs = jax.nn.sigmoid(_dot(h2, wd_ref[...]) + bd_ref[...])
    lo_ref[...] = preds - diffs
    hi_ref[...] = preds + diffs


def _tc_combine2(q0, q1, c0, c1, W_l2, r2, W_pred, b_pred, W_diff, b_diff):
    return pl.pallas_call(
        _combine2_body,
        grid=(_GRID,),
        in_specs=[
            pl.BlockSpec((_RB, D), lambda i: (i, 0)),
            pl.BlockSpec((_RB, D), lambda i: (i, 0)),
            pl.BlockSpec((_RB, CNTW), lambda i: (i, 0)),
            pl.BlockSpec((_RB, CNTW), lambda i: (i, 0)),
            pl.BlockSpec((D, D), lambda i: (0, 0)),
            pl.BlockSpec((_RB, D), lambda i: (i, 0)),
            pl.BlockSpec((D, 1), lambda i: (0, 0)),
            pl.BlockSpec((1,), lambda i: (0,)),
            pl.BlockSpec((D, 1), lambda i: (0, 0)),
            pl.BlockSpec((1,), lambda i: (0,)),
        ],
        out_specs=[
            pl.BlockSpec((_RB, 1), lambda i: (i, 0)),
            pl.BlockSpec((_RB, 1), lambda i: (i, 0)),
        ],
        out_shape=[
            jax.ShapeDtypeStruct((N, 1), jnp.float32),
            jax.ShapeDtypeStruct((N, 1), jnp.float32),
        ],
    )(q0, q1, c0, c1, W_l2, r2, W_pred, b_pred, W_diff, b_diff)


def kernel(x, W_l1, b_l1, W_r1, W_l2, b_l2, W_r2, W_pred, b_pred, W_diff,
           b_diff, edge_index):
    pad_src = jnp.zeros((E_PAD - E,), jnp.int32)
    pad_dst = jnp.full((E_PAD - E,), N_PAD - 1, jnp.int32)
    src = jnp.concatenate([edge_index[0], pad_src]).reshape(NROW, CHUNK)
    dst = jnp.concatenate([edge_index[1], pad_dst]).reshape(NROW, CHUNK)

    (agg1,) = _sc_agg(x, src, dst)
    (cnt,) = _sc_counts(dst)
    r1 = _tc_linear(x, W_r1, b_l1)

    a0, a1 = agg1[:N], agg1[N_PAD:N_PAD + N]
    c0, c1 = cnt[:N], cnt[N_PAD:N_PAD + N]
    h1 = _tc_h1(a0, a1, c0, c1, W_l1, r1)

    (agg2,) = _sc_agg(h1, src, dst)
    r2 = _tc_linear(h1, W_r2, b_l2)
    q0, q1 = agg2[:N], agg2[N_PAD:N_PAD + N]
    preds_low, preds_upper = _tc_combine2(q0, q1, c0, c1, W_l2, r2, W_pred,
                                          b_pred, W_diff, b_diff)
    return (preds_low, preds_upper)


# speedup vs baseline: 2.6859x; 2.6859x over previous
import functools

import jax
import jax.numpy as jnp
from jax import lax
from jax.experimental import pallas as pl
from jax.experimental.pallas import tpu as pltpu
from jax.experimental.pallas import tpu_sc as plsc

N = 10000
E = 320000
D = 128
NC = 2
NS = 16
CHUNK = 128
E_PAD = 327680
NCH_T = 80
NROW = E_PAD // CHUNK
N_PAD = 10240
SLICE = N_PAD // NS
ZB = 64
CNTW = 128
G = 8
NCHG = 16

_mesh = plsc.VectorSubcoreMesh(core_axis_name="c", subcore_axis_name="s")


def _fill2d(ref, rows, cols, value):
    val = jnp.full((1, 16), value, jnp.float32)

    @pl.loop(0, rows)
    def _(r):
        @pl.loop(0, cols, step=16)
        def _(j):
            ref[pl.ds(r, 1), pl.ds(j, 16)] = val


def _make_sc_pass():
    out_types = [jax.ShapeDtypeStruct((NC * N_PAD, D), jnp.float32)]
    scratch = [
        pltpu.VMEM_SHARED((N_PAD, D), jnp.float32),
        pltpu.VMEM((NCHG, CHUNK), jnp.int32),
        pltpu.VMEM((NCHG, CHUNK), jnp.int32),
        pltpu.VMEM((CHUNK, D), jnp.float32),
        pltpu.VMEM((CHUNK, D), jnp.float32),
        pltpu.VMEM((ZB, D), jnp.float32),
        pltpu.SemaphoreType.DMA,
        pltpu.SemaphoreType.DMA,
        pltpu.SemaphoreType.DMA,
        pltpu.SemaphoreType.DMA,
    ]

    def body(table_hbm, src_hbm, dst_hbm, acc_out, acc_sh, isrc, idst,
             rows0, rows1, zero_v, gs0, gs1, ss0, ss1):
        c = lax.axis_index("c")
        s = lax.axis_index("s")
        wid = c * NS + s

        _fill2d(zero_v, ZB, D, 0.0)
        tile_base = s * SLICE

        @pl.loop(0, SLICE, step=ZB)
        def _(r):
            pltpu.sync_copy(zero_v, acc_sh.at[pl.ds(tile_base + r, ZB)])

        plsc.subcore_barrier()

        @pl.loop(0, NCH_T // NCHG)
        def _(g):
            row0 = wid * NCH_T + g * NCHG
            pltpu.sync_copy(src_hbm.at[pl.ds(row0, NCHG)], isrc)
            pltpu.sync_copy(dst_hbm.at[pl.ds(row0, NCHG)], idst)
            pltpu.async_copy(table_hbm.at[isrc.at[0]], rows0, gs0)

            @pl.loop(0, NCHG, step=2)
            def _(k):
                @pl.when(k > 0)
                def _():
                    pltpu.make_async_copy(
                        rows1, acc_sh.at[idst.at[k - 1]], ss1).wait()

                @pl.when(k + 1 < NCHG)
                def _():
                    pltpu.async_copy(table_hbm.at[isrc.at[k + 1]], rows1, gs1)

                pltpu.make_async_copy(
                    table_hbm.at[isrc.at[k]], rows0, gs0).wait()
                pltpu.async_copy(rows0, acc_sh.at[idst.at[k]], ss0, add=True)

                pltpu.make_async_copy(
                    rows0, acc_sh.at[idst.at[k]], ss0).wait()

                @pl.when(k + 2 < NCHG)
                def _():
                    pltpu.async_copy(table_hbm.at[isrc.at[k + 2]], rows0, gs0)

                pltpu.make_async_copy(
                    table_hbm.at[isrc.at[k + 1]], rows1, gs1).wait()
                pltpu.async_copy(rows1, acc_sh.at[idst.at[k + 1]], ss1,
                                 add=True)

            pltpu.make_async_copy(
                rows1, acc_sh.at[idst.at[NCHG - 1]], ss1).wait()

        plsc.subcore_barrier()

        pltpu.sync_copy(acc_sh.at[pl.ds(tile_base, SLICE)],
                        acc_out.at[pl.ds(c * N_PAD + tile_base, SLICE)])

    return pl.kernel(body, out_type=out_types, mesh=_mesh,
                     scratch_types=scratch)


def _make_sc_counts():
    out_types = [jax.ShapeDtypeStruct((NC * N_PAD, CNTW), jnp.float32)]
    scratch = [
        pltpu.VMEM_SHARED((N_PAD, CNTW), jnp.float32),
        pltpu.VMEM((G, CHUNK), jnp.int32),
        pltpu.VMEM((CHUNK, CNTW), jnp.float32),
        pltpu.VMEM((ZB, CNTW), jnp.float32),
        pltpu.SemaphoreType.DMA,
    ]

    def body(dst_hbm, cnt_out, cnt_sh, idx_dst, ones_v, zero_v, sem):
        c = lax.axis_index("c")
        s = lax.axis_index("s")
        wid = c * NS + s

        _fill2d(zero_v, ZB, CNTW, 0.0)
        _fill2d(ones_v, CHUNK, CNTW, 1.0)
        tile_base = s * SLICE

        @pl.loop(0, SLICE, step=ZB)
        def _(r):
            pltpu.sync_copy(zero_v, cnt_sh.at[pl.ds(tile_base + r, ZB)])

        plsc.subcore_barrier()

        @pl.loop(0, NCH_T // G)
        def _(g):
            row0 = wid * NCH_T + g * G
            pltpu.sync_copy(dst_hbm.at[pl.ds(row0, G)], idx_dst)

            @pl.loop(0, G)
            def _(i):
                pltpu.sync_copy(ones_v, cnt_sh.at[idx_dst.at[i]], add=True)

        plsc.subcore_barrier()

        pltpu.sync_copy(cnt_sh.at[pl.ds(tile_base, SLICE)],
                        cnt_out.at[pl.ds(c * N_PAD + tile_base, SLICE)])

    return pl.kernel(body, out_type=out_types, mesh=_mesh,
                     scratch_types=scratch)


_sc_agg = _make_sc_pass()
_sc_counts = _make_sc_counts()


_RB = 1000
_GRID = N // _RB


def _dot(a, b):
    return jax.lax.dot_general(a, b, (((1,), (0,)), ((), ())),
                               preferred_element_type=jnp.float32,
                               precision=jax.lax.Precision.HIGHEST)


def _lin_body(x_ref, w_ref, b_ref, o_ref):
    o_ref[...] = _dot(x_ref[...], w_ref[...]) + b_ref[...]


def _tc_linear(x, w, b):
    return pl.pallas_call(
        _lin_body,
        grid=(_GRID,),
        in_specs=[
            pl.BlockSpec((_RB, D), lambda i: (i, 0)),
            pl.BlockSpec((D, D), lambda i: (0, 0)),
            pl.BlockSpec((D,), lambda i: (0,)),
        ],
        out_specs=pl.BlockSpec((_RB, D), lambda i: (i, 0)),
        out_shape=jax.ShapeDtypeStruct((N, D), jnp.float32),
    )(x, w, b)


def _h1_body(a0_ref, a1_ref, c0_ref, c1_ref, wl_ref, r1_ref, h1_ref):
    cnt = c0_ref[...][:, 0:1] + c1_ref[...][:, 0:1]
    mean = (a0_ref[...] + a1_ref[...]) / jnp.maximum(cnt, 1.0)
    h1_ref[...] = jax.nn.relu(_dot(mean, wl_ref[...]) + r1_ref[...])


def _tc_h1(a0, a1, c0, c1, W_l1, r1):
    return pl.pallas_call(
        _h1_body,
        grid=(_GRID,),
        in_specs=[
            pl.BlockSpec((_RB, D), lambda i: (i, 0)),
            pl.BlockSpec((_RB, D), lambda i: (i, 0)),
            pl.BlockSpec((_RB, CNTW), lambda i: (i, 0)),
            pl.BlockSpec((_RB, CNTW), lambda i: (i, 0)),
            pl.BlockSpec((D, D), lambda i: (0, 0)),
            pl.BlockSpec((_RB, D), lambda i: (i, 0)),
        ],
        out_specs=pl.BlockSpec((_RB, D), lambda i: (i, 0)),
        out_shape=jax.ShapeDtypeStruct((N, D), jnp.float32),
    )(a0, a1, c0, c1, W_l1, r1)


def _combine2_body(q0_ref, q1_ref, c0_ref, c1_ref, wl_ref, r2_ref, wp_ref,
                   bp_ref, wd_ref, bd_ref, lo_ref, hi_ref):
    cnt = c0_ref[...][:, 0:1] + c1_ref[...][:, 0:1]
    mean = (q0_ref[...] + q1_ref[...]) / jnp.maximum(cnt, 1.0)
    h2 = jax.nn.relu(_dot(mean, wl_ref[...]) + r2_ref[...])
    preds = _dot(h2, wp_ref[...]) + bp_ref[...]
    diffs = jax.nn.sigmoid(_dot(h2, wd_ref[...]) + bd_ref[...])
    lo_ref[...] = preds - diffs
    hi_ref[...] = preds + diffs


def _tc_combine2(q0, q1, c0, c1, W_l2, r2, W_pred, b_pred, W_diff, b_diff):
    return pl.pallas_call(
        _combine2_body,
        grid=(_GRID,),
        in_specs=[
            pl.BlockSpec((_RB, D), lambda i: (i, 0)),
            pl.BlockSpec((_RB, D), lambda i: (i, 0)),
            pl.BlockSpec((_RB, CNTW), lambda i: (i, 0)),
            pl.BlockSpec((_RB, CNTW), lambda i: (i, 0)),
            pl.BlockSpec((D, D), lambda i: (0, 0)),
            pl.BlockSpec((_RB, D), lambda i: (i, 0)),
            pl.BlockSpec((D, 1), lambda i: (0, 0)),
            pl.BlockSpec((1,), lambda i: (0,)),
            pl.BlockSpec((D, 1), lambda i: (0, 0)),
            pl.BlockSpec((1,), lambda i: (0,)),
        ],
        out_specs=[
            pl.BlockSpec((_RB, 1), lambda i: (i, 0)),
            pl.BlockSpec((_RB, 1), lambda i: (i, 0)),
        ],
        out_shape=[
            jax.ShapeDtypeStruct((N, 1), jnp.float32),
            jax.ShapeDtypeStruct((N, 1), jnp.float32),
        ],
    )(q0, q1, c0, c1, W_l2, r2, W_pred, b_pred, W_diff, b_diff)


def kernel(x, W_l1, b_l1, W_r1, W_l2, b_l2, W_r2, W_pred, b_pred, W_diff,
           b_diff, edge_index):
    pad_src = jnp.arange(E_PAD - E, dtype=jnp.int32) % N
    pad_dst = N + jnp.arange(E_PAD - E, dtype=jnp.int32) % (N_PAD - N)
    src = jnp.concatenate([edge_index[0], pad_src]).reshape(NROW, CHUNK)
    dst = jnp.concatenate([edge_index[1], pad_dst]).reshape(NROW, CHUNK)

    (agg1,) = _sc_agg(x, src, dst)
    (cnt,) = _sc_counts(dst)
    r1 = _tc_linear(x, W_r1, b_l1)

    a0, a1 = agg1[:N], agg1[N_PAD:N_PAD + N]
    c0, c1 = cnt[:N], cnt[N_PAD:N_PAD + N]
    h1 = _tc_h1(a0, a1, c0, c1, W_l1, r1)

    (agg2,) = _sc_agg(h1, src, dst)
    r2 = _tc_linear(h1, W_r2, b_l2)
    q0, q1 = agg2[:N], agg2[N_PAD:N_PAD + N]
    preds_low, preds_upper = _tc_combine2(q0, q1, c0, c1, W_l2, r2, W_pred,
                                          b_pred, W_diff, b_diff)
    return (preds_low, preds_upper)


# speedup vs baseline: 2.7064x; 1.0076x over previous
import functools

import jax
import jax.numpy as jnp
from jax import lax
from jax.experimental import pallas as pl
from jax.experimental.pallas import tpu as pltpu
from jax.experimental.pallas import tpu_sc as plsc

N = 10000
E = 320000
D = 128
NC = 2
NS = 16
CHUNK = 128
E_PAD = 327680
NCH_T = 80
NROW = E_PAD // CHUNK
N_PAD = 10240
SLICE = N_PAD // NS
ZB = 64
CNTW = 128
G = 8
NCHG = 16

_mesh = plsc.VectorSubcoreMesh(core_axis_name="c", subcore_axis_name="s")


def _fill2d(ref, rows, cols, value):
    val = jnp.full((1, 16), value, jnp.float32)

    @pl.loop(0, rows)
    def _(r):
        @pl.loop(0, cols, step=16)
        def _(j):
            ref[pl.ds(r, 1), pl.ds(j, 16)] = val


def _make_sc_pass():
    out_types = [jax.ShapeDtypeStruct((NC * N_PAD, D), jnp.float32)]
    scratch = [
        pltpu.VMEM_SHARED((N_PAD, D), jnp.float32),
        pltpu.VMEM((NCHG, CHUNK), jnp.int32),
        pltpu.VMEM((NCHG, CHUNK), jnp.int32),
        pltpu.VMEM((CHUNK, D), jnp.float32),
        pltpu.VMEM((CHUNK, D), jnp.float32),
        pltpu.VMEM((ZB, D), jnp.float32),
        pltpu.SemaphoreType.DMA,
        pltpu.SemaphoreType.DMA,
        pltpu.SemaphoreType.DMA,
        pltpu.SemaphoreType.DMA,
    ]

    def body(table_hbm, src_hbm, dst_hbm, acc_out, acc_sh, isrc, idst,
             rows0, rows1, zero_v, gs0, gs1, ss0, ss1):
        c = lax.axis_index("c")
        s = lax.axis_index("s")
        wid = c * NS + s

        _fill2d(zero_v, ZB, D, 0.0)
        tile_base = s * SLICE

        @pl.loop(0, SLICE, step=ZB)
        def _(r):
            pltpu.sync_copy(zero_v, acc_sh.at[pl.ds(tile_base + r, ZB)])

        plsc.subcore_barrier()

        @pl.loop(0, NCH_T // NCHG)
        def _(g):
            row0 = wid * NCH_T + g * NCHG
            pltpu.sync_copy(src_hbm.at[pl.ds(row0, NCHG)], isrc)
            pltpu.sync_copy(dst_hbm.at[pl.ds(row0, NCHG)], idst)
            pltpu.async_copy(table_hbm.at[isrc.at[0]], rows0, gs0)

            @pl.loop(0, NCHG, step=2)
            def _(k):
                @pl.when(k > 0)
                def _():
                    pltpu.make_async_copy(
                        rows1, acc_sh.at[idst.at[k - 1]], ss1).wait()

                @pl.when(k + 1 < NCHG)
                def _():
                    pltpu.async_copy(table_hbm.at[isrc.at[k + 1]], rows1, gs1)

                pltpu.make_async_copy(
                    table_hbm.at[isrc.at[k]], rows0, gs0).wait()
                pltpu.async_copy(rows0, acc_sh.at[idst.at[k]], ss0, add=True)

                pltpu.make_async_copy(
                    rows0, acc_sh.at[idst.at[k]], ss0).wait()

                @pl.when(k + 2 < NCHG)
                def _():
                    pltpu.async_copy(table_hbm.at[isrc.at[k + 2]], rows0, gs0)

                pltpu.make_async_copy(
                    table_hbm.at[isrc.at[k + 1]], rows1, gs1).wait()
                pltpu.async_copy(rows1, acc_sh.at[idst.at[k + 1]], ss1,
                                 add=True)

            pltpu.make_async_copy(
                rows1, acc_sh.at[idst.at[NCHG - 1]], ss1).wait()

        plsc.subcore_barrier()

        pltpu.sync_copy(acc_sh.at[pl.ds(tile_base, SLICE)],
                        acc_out.at[pl.ds(c * N_PAD + tile_base, SLICE)])

    return pl.kernel(body, out_type=out_types, mesh=_mesh,
                     scratch_types=scratch)


def _make_sc_counts():
    out_types = [jax.ShapeDtypeStruct((NC * N_PAD, CNTW), jnp.float32)]
    scratch = [
        pltpu.VMEM_SHARED((N_PAD, CNTW), jnp.float32),
        pltpu.VMEM((NCHG, CHUNK), jnp.int32),
        pltpu.VMEM((CHUNK, CNTW), jnp.float32),
        pltpu.VMEM((ZB, CNTW), jnp.float32),
        pltpu.SemaphoreType.DMA,
    ]

    def body(dst_hbm, cnt_out, cnt_sh, idx_dst, ones_v, zero_v, sem):
        c = lax.axis_index("c")
        s = lax.axis_index("s")
        wid = c * NS + s

        _fill2d(zero_v, ZB, CNTW, 0.0)
        _fill2d(ones_v, CHUNK, CNTW, 1.0)
        tile_base = s * SLICE

        @pl.loop(0, SLICE, step=ZB)
        def _(r):
            pltpu.sync_copy(zero_v, cnt_sh.at[pl.ds(tile_base + r, ZB)])

        plsc.subcore_barrier()

        @pl.loop(0, NCH_T // NCHG)
        def _(g):
            row0 = wid * NCH_T + g * NCHG
            pltpu.sync_copy(dst_hbm.at[pl.ds(row0, NCHG)], idx_dst)

            @pl.loop(0, NCHG)
            def _(i):
                pltpu.async_copy(ones_v, cnt_sh.at[idx_dst.at[i]], sem,
                                 add=True)

            @pl.loop(0, NCHG)
            def _(i):
                pltpu.make_async_copy(ones_v, cnt_sh.at[idx_dst.at[i]],
                                      sem).wait()

        plsc.subcore_barrier()

        pltpu.sync_copy(cnt_sh.at[pl.ds(tile_base, SLICE)],
                        cnt_out.at[pl.ds(c * N_PAD + tile_base, SLICE)])

    return pl.kernel(body, out_type=out_types, mesh=_mesh,
                     scratch_types=scratch)


_sc_agg = _make_sc_pass()
_sc_counts = _make_sc_counts()


_RB = 1000
_GRID = N // _RB


def _dot(a, b):
    return jax.lax.dot_general(a, b, (((1,), (0,)), ((), ())),
                               preferred_element_type=jnp.float32,
                               precision=jax.lax.Precision.HIGHEST)


def _lin_body(x_ref, w_ref, b_ref, o_ref):
    o_ref[...] = _dot(x_ref[...], w_ref[...]) + b_ref[...]


def _tc_linear(x, w, b):
    return pl.pallas_call(
        _lin_body,
        grid=(_GRID,),
        in_specs=[
            pl.BlockSpec((_RB, D), lambda i: (i, 0)),
            pl.BlockSpec((D, D), lambda i: (0, 0)),
            pl.BlockSpec((D,), lambda i: (0,)),
        ],
        out_specs=pl.BlockSpec((_RB, D), lambda i: (i, 0)),
        out_shape=jax.ShapeDtypeStruct((N, D), jnp.float32),
    )(x, w, b)


def _h1_body(a0_ref, a1_ref, c0_ref, c1_ref, wl_ref, r1_ref, h1_ref):
    cnt = c0_ref[...][:, 0:1] + c1_ref[...][:, 0:1]
    mean = (a0_ref[...] + a1_ref[...]) / jnp.maximum(cnt, 1.0)
    h1_ref[...] = jax.nn.relu(_dot(mean, wl_ref[...]) + r1_ref[...])


def _tc_h1(a0, a1, c0, c1, W_l1, r1):
    return pl.pallas_call(
        _h1_body,
        grid=(_GRID,),
        in_specs=[
            pl.BlockSpec((_RB, D), lambda i: (i, 0)),
            pl.BlockSpec((_RB, D), lambda i: (i, 0)),
            pl.BlockSpec((_RB, 16), lambda i: (i, 0)),
            pl.BlockSpec((_RB, 16), lambda i: (i, 0)),
            pl.BlockSpec((D, D), lambda i: (0, 0)),
            pl.BlockSpec((_RB, D), lambda i: (i, 0)),
        ],
        out_specs=pl.BlockSpec((_RB, D), lambda i: (i, 0)),
        out_shape=jax.ShapeDtypeStruct((N, D), jnp.float32),
    )(a0, a1, c0, c1, W_l1, r1)


def _combine2_body(q0_ref, q1_ref, c0_ref, c1_ref, wl_ref, r2_ref, wp_ref,
                   bp_ref, wd_ref, bd_ref, lo_ref, hi_ref):
    cnt = c0_ref[...][:, 0:1] + c1_ref[...][:, 0:1]
    mean = (q0_ref[...] + q1_ref[...]) / jnp.maximum(cnt, 1.0)
    h2 = jax.nn.relu(_dot(mean, wl_ref[...]) + r2_ref[...])
    preds = _dot(h2, wp_ref[...]) + bp_ref[...]
    diffs = jax.nn.sigmoid(_dot(h2, wd_ref[...]) + bd_ref[...])
    lo_ref[...] = preds - diffs
    hi_ref[...] = preds + diffs


def _tc_combine2(q0, q1, c0, c1, W_l2, r2, W_pred, b_pred, W_diff, b_diff):
    return pl.pallas_call(
        _combine2_body,
        grid=(_GRID,),
        in_specs=[
            pl.BlockSpec((_RB, D), lambda i: (i, 0)),
            pl.BlockSpec((_RB, D), lambda i: (i, 0)),
            pl.BlockSpec((_RB, 16), lambda i: (i, 0)),
            pl.BlockSpec((_RB, 16), lambda i: (i, 0)),
            pl.BlockSpec((D, D), lambda i: (0, 0)),
            pl.BlockSpec((_RB, D), lambda i: (i, 0)),
            pl.BlockSpec((D, 1), lambda i: (0, 0)),
            pl.BlockSpec((1,), lambda i: (0,)),
            pl.BlockSpec((D, 1), lambda i: (0, 0)),
            pl.BlockSpec((1,), lambda i: (0,)),
        ],
        out_specs=[
            pl.BlockSpec((_RB, 1), lambda i: (i, 0)),
            pl.BlockSpec((_RB, 1), lambda i: (i, 0)),
        ],
        out_shape=[
            jax.ShapeDtypeStruct((N, 1), jnp.float32),
            jax.ShapeDtypeStruct((N, 1), jnp.float32),
        ],
    )(q0, q1, c0, c1, W_l2, r2, W_pred, b_pred, W_diff, b_diff)


def kernel(x, W_l1, b_l1, W_r1, W_l2, b_l2, W_r2, W_pred, b_pred, W_diff,
           b_diff, edge_index):
    pad_src = jnp.arange(E_PAD - E, dtype=jnp.int32) % N
    pad_dst = N + jnp.arange(E_PAD - E, dtype=jnp.int32) % (N_PAD - N)
    src = jnp.concatenate([edge_index[0], pad_src]).reshape(NROW, CHUNK)
    dst = jnp.concatenate([edge_index[1], pad_dst]).reshape(NROW, CHUNK)

    (agg1,) = _sc_agg(x, src, dst)
    (cnt,) = _sc_counts(dst)
    r1 = _tc_linear(x, W_r1, b_l1)

    a0, a1 = agg1[:N], agg1[N_PAD:N_PAD + N]
    c0 = cnt[:N, :16]
    c1 = cnt[N_PAD:N_PAD + N, :16]
    h1 = _tc_h1(a0, a1, c0, c1, W_l1, r1)

    (agg2,) = _sc_agg(h1, src, dst)
    r2 = _tc_linear(h1, W_r2, b_l2)
    q0, q1 = agg2[:N], agg2[N_PAD:N_PAD + N]
    preds_low, preds_upper = _tc_combine2(q0, q1, c0, c1, W_l2, r2, W_pred,
                                          b_pred, W_diff, b_diff)
    return (preds_low, preds_upper)


# speedup vs baseline: 2.8124x; 1.0392x over previous
import functools

import jax
import jax.numpy as jnp
from jax import lax
from jax.experimental import pallas as pl
from jax.experimental.pallas import tpu as pltpu
from jax.experimental.pallas import tpu_sc as plsc

N = 10000
E = 320000
D = 128
NC = 2
NS = 16
CHUNK = 128
E_PAD = 327680
NCH_T = 80
NROW = E_PAD // CHUNK
N_PAD = 10240
SLICE = N_PAD // NS
ZB = 64
CNTW = 128
G = 8
NCHG = 16

_mesh = plsc.VectorSubcoreMesh(core_axis_name="c", subcore_axis_name="s")


def _fill2d(ref, rows, cols, value):
    val = jnp.full((1, 16), value, jnp.float32)

    @pl.loop(0, rows)
    def _(r):
        @pl.loop(0, cols, step=16)
        def _(j):
            ref[pl.ds(r, 1), pl.ds(j, 16)] = val


def _make_sc_pass():
    out_types = [jax.ShapeDtypeStruct((NC * N_PAD, D), jnp.float32)]
    scratch = [
        pltpu.VMEM_SHARED((N_PAD, D), jnp.float32),
        pltpu.VMEM((NCHG, CHUNK), jnp.int32),
        pltpu.VMEM((NCHG, CHUNK), jnp.int32),
        pltpu.VMEM((CHUNK, D), jnp.float32),
        pltpu.VMEM((CHUNK, D), jnp.float32),
        pltpu.VMEM((ZB, D), jnp.float32),
        pltpu.SemaphoreType.DMA,
        pltpu.SemaphoreType.DMA,
        pltpu.SemaphoreType.DMA,
        pltpu.SemaphoreType.DMA,
    ]

    def body(table_hbm, src_hbm, dst_hbm, acc_out, acc_sh, isrc, idst,
             rows0, rows1, zero_v, gs0, gs1, ss0, ss1):
        c = lax.axis_index("c")
        s = lax.axis_index("s")
        wid = c * NS + s

        _fill2d(zero_v, ZB, D, 0.0)
        tile_base = s * SLICE

        @pl.loop(0, SLICE, step=ZB)
        def _(r):
            pltpu.sync_copy(zero_v, acc_sh.at[pl.ds(tile_base + r, ZB)])

        plsc.subcore_barrier()

        @pl.loop(0, NCH_T // NCHG)
        def _(g):
            row0 = wid * NCH_T + g * NCHG
            pltpu.sync_copy(src_hbm.at[pl.ds(row0, NCHG)], isrc)
            pltpu.sync_copy(dst_hbm.at[pl.ds(row0, NCHG)], idst)
            pltpu.async_copy(table_hbm.at[isrc.at[0]], rows0, gs0)

            @pl.loop(0, NCHG, step=2)
            def _(k):
                @pl.when(k > 0)
                def _():
                    pltpu.make_async_copy(
                        rows1, acc_sh.at[idst.at[k - 1]], ss1).wait()

                @pl.when(k + 1 < NCHG)
                def _():
                    pltpu.async_copy(table_hbm.at[isrc.at[k + 1]], rows1, gs1)

                pltpu.make_async_copy(
                    table_hbm.at[isrc.at[k]], rows0, gs0).wait()
                pltpu.async_copy(rows0, acc_sh.at[idst.at[k]], ss0, add=True)

                pltpu.make_async_copy(
                    rows0, acc_sh.at[idst.at[k]], ss0).wait()

                @pl.when(k + 2 < NCHG)
                def _():
                    pltpu.async_copy(table_hbm.at[isrc.at[k + 2]], rows0, gs0)

                pltpu.make_async_copy(
                    table_hbm.at[isrc.at[k + 1]], rows1, gs1).wait()
                pltpu.async_copy(rows1, acc_sh.at[idst.at[k + 1]], ss1,
                                 add=True)

            pltpu.make_async_copy(
                rows1, acc_sh.at[idst.at[NCHG - 1]], ss1).wait()

        plsc.subcore_barrier()

        pltpu.sync_copy(acc_sh.at[pl.ds(tile_base, SLICE)],
                        acc_out.at[pl.ds(c * N_PAD + tile_base, SLICE)])

    return pl.kernel(body, out_type=out_types, mesh=_mesh,
                     scratch_types=scratch)


def _make_sc_counts():
    out_types = [jax.ShapeDtypeStruct((NC * N_PAD, CNTW), jnp.float32)]
    scratch = [
        pltpu.VMEM_SHARED((N_PAD, CNTW), jnp.float32),
        pltpu.VMEM((NCHG, CHUNK), jnp.int32),
        pltpu.VMEM((CHUNK, CNTW), jnp.float32),
        pltpu.VMEM((ZB, CNTW), jnp.float32),
        pltpu.SemaphoreType.DMA,
    ]

    def body(dst_hbm, cnt_out, cnt_sh, idx_dst, ones_v, zero_v, sem):
        c = lax.axis_index("c")
        s = lax.axis_index("s")
        wid = c * NS + s

        _fill2d(zero_v, ZB, CNTW, 0.0)
        _fill2d(ones_v, CHUNK, CNTW, 1.0)
        tile_base = s * SLICE

        @pl.loop(0, SLICE, step=ZB)
        def _(r):
            pltpu.sync_copy(zero_v, cnt_sh.at[pl.ds(tile_base + r, ZB)])

        plsc.subcore_barrier()

        @pl.loop(0, NCH_T // NCHG)
        def _(g):
            row0 = wid * NCH_T + g * NCHG
            pltpu.sync_copy(dst_hbm.at[pl.ds(row0, NCHG)], idx_dst)

            @pl.loop(0, NCHG)
            def _(i):
                pltpu.async_copy(ones_v, cnt_sh.at[idx_dst.at[i]], sem,
                                 add=True)

            @pl.loop(0, NCHG)
            def _(i):
                pltpu.make_async_copy(ones_v, cnt_sh.at[idx_dst.at[i]],
                                      sem).wait()

        plsc.subcore_barrier()

        pltpu.sync_copy(cnt_sh.at[pl.ds(tile_base, SLICE)],
                        cnt_out.at[pl.ds(c * N_PAD + tile_base, SLICE)])

    return pl.kernel(body, out_type=out_types, mesh=_mesh,
                     scratch_types=scratch)


_sc_agg = _make_sc_pass()
_sc_counts = _make_sc_counts()


_RB = 1000
_GRID = N // _RB


def _dot(a, b):
    return jax.lax.dot_general(a, b, (((1,), (0,)), ((), ())),
                               preferred_element_type=jnp.float32,
                               precision=jax.lax.Precision.DEFAULT)


def _lin_body(x_ref, w_ref, b_ref, o_ref):
    o_ref[...] = _dot(x_ref[...], w_ref[...]) + b_ref[...]


def _tc_linear(x, w, b):
    return pl.pallas_call(
        _lin_body,
        grid=(_GRID,),
        in_specs=[
            pl.BlockSpec((_RB, D), lambda i: (i, 0)),
            pl.BlockSpec((D, D), lambda i: (0, 0)),
            pl.BlockSpec((D,), lambda i: (0,)),
        ],
        out_specs=pl.BlockSpec((_RB, D), lambda i: (i, 0)),
        out_shape=jax.ShapeDtypeStruct((N, D), jnp.float32),
    )(x, w, b)


def _h1_body(a0_ref, a1_ref, c0_ref, c1_ref, wl_ref, r1_ref, h1_ref):
    cnt = c0_ref[...][:, 0:1] + c1_ref[...][:, 0:1]
    mean = (a0_ref[...] + a1_ref[...]) / jnp.maximum(cnt, 1.0)
    h1_ref[...] = jax.nn.relu(_dot(mean, wl_ref[...]) + r1_ref[...])


def _tc_h1(a0, a1, c0, c1, W_l1, r1):
    return pl.pallas_call(
        _h1_body,
        grid=(_GRID,),
        in_specs=[
            pl.BlockSpec((_RB, D), lambda i: (i, 0)),
            pl.BlockSpec((_RB, D), lambda i: (i, 0)),
            pl.BlockSpec((_RB, 16), lambda i: (i, 0)),
            pl.BlockSpec((_RB, 16), lambda i: (i, 0)),
            pl.BlockSpec((D, D), lambda i: (0, 0)),
            pl.BlockSpec((_RB, D), lambda i: (i, 0)),
        ],
        out_specs=pl.BlockSpec((_RB, D), lambda i: (i, 0)),
        out_shape=jax.ShapeDtypeStruct((N, D), jnp.float32),
    )(a0, a1, c0, c1, W_l1, r1)


def _combine2_body(q0_ref, q1_ref, c0_ref, c1_ref, wl_ref, r2_ref, wp_ref,
                   bp_ref, wd_ref, bd_ref, lo_ref, hi_ref):
    cnt = c0_ref[...][:, 0:1] + c1_ref[...][:, 0:1]
    mean = (q0_ref[...] + q1_ref[...]) / jnp.maximum(cnt, 1.0)
    h2 = jax.nn.relu(_dot(mean, wl_ref[...]) + r2_ref[...])
    preds = _dot(h2, wp_ref[...]) + bp_ref[...]
    diffs = jax.nn.sigmoid(_dot(h2, wd_ref[...]) + bd_ref[...])
    lo_ref[...] = preds - diffs
    hi_ref[...] = preds + diffs


def _tc_combine2(q0, q1, c0, c1, W_l2, r2, W_pred, b_pred, W_diff, b_diff):
    return pl.pallas_call(
        _combine2_body,
        grid=(_GRID,),
        in_specs=[
            pl.BlockSpec((_RB, D), lambda i: (i, 0)),
            pl.BlockSpec((_RB, D), lambda i: (i, 0)),
            pl.BlockSpec((_RB, 16), lambda i: (i, 0)),
            pl.BlockSpec((_RB, 16), lambda i: (i, 0)),
            pl.BlockSpec((D, D), lambda i: (0, 0)),
            pl.BlockSpec((_RB, D), lambda i: (i, 0)),
            pl.BlockSpec((D, 1), lambda i: (0, 0)),
            pl.BlockSpec((1,), lambda i: (0,)),
            pl.BlockSpec((D, 1), lambda i: (0, 0)),
            pl.BlockSpec((1,), lambda i: (0,)),
        ],
        out_specs=[
            pl.BlockSpec((_RB, 1), lambda i: (i, 0)),
            pl.BlockSpec((_RB, 1), lambda i: (i, 0)),
        ],
        out_shape=[
            jax.ShapeDtypeStruct((N, 1), jnp.float32),
            jax.ShapeDtypeStruct((N, 1), jnp.float32),
        ],
    )(q0, q1, c0, c1, W_l2, r2, W_pred, b_pred, W_diff, b_diff)


def kernel(x, W_l1, b_l1, W_r1, W_l2, b_l2, W_r2, W_pred, b_pred, W_diff,
           b_diff, edge_index):
    pad_src = jnp.arange(E_PAD - E, dtype=jnp.int32) % N
    pad_dst = N + jnp.arange(E_PAD - E, dtype=jnp.int32) % (N_PAD - N)
    src = jnp.concatenate([edge_index[0], pad_src]).reshape(NROW, CHUNK)
    dst = jnp.concatenate([edge_index[1], pad_dst]).reshape(NROW, CHUNK)

    (agg1,) = _sc_agg(x, src, dst)
    (cnt,) = _sc_counts(dst)
    r1 = _tc_linear(x, W_r1, b_l1)

    a0, a1 = agg1[:N], agg1[N_PAD:N_PAD + N]
    c0 = cnt[:N, :16]
    c1 = cnt[N_PAD:N_PAD + N, :16]
    h1 = _tc_h1(a0, a1, c0, c1, W_l1, r1)

    (agg2,) = _sc_agg(h1, src, dst)
    r2 = _tc_linear(h1, W_r2, b_l2)
    q0, q1 = agg2[:N], agg2[N_PAD:N_PAD + N]
    preds_low, preds_upper = _tc_combine2(q0, q1, c0, c1, W_l2, r2, W_pred,
                                          b_pred, W_diff, b_diff)
    return (preds_low, preds_upper)
